# baseline (speedup 1.0000x reference)
import jax
import jax.numpy as jnp
from jax.experimental import pallas as pl
from jax.experimental.pallas import tpu as pltpu

_NORM = 100.0
_THRESH2 = 100.0


def _silu(x):
    h = 0.5 * x
    return h * jnp.tanh(h) + h


def _dot(a, b, prec=jax.lax.Precision.DEFAULT, out_t=jnp.float32):
    return jax.lax.dot_general(
        a, b, (((a.ndim - 1,), (0,)), ((), ())),
        preferred_element_type=out_t, precision=prec)


def _hp_proj_body(hp_ref, Wa1_ref, Wv1_ref, bv1_ref, Wc1_ref,
                  hpA_ref, hpC_ref, hpV_ref):
    lig_nf = Wa1_ref.shape[0] - hp_ref.shape[2] - 1
    prot_nf = hp_ref.shape[2]
    hp = hp_ref[0]
    hpA_ref[0] = _dot(hp, Wa1_ref[lig_nf:lig_nf + prot_nf, :]).astype(jnp.bfloat16)
    hpC_ref[0] = _dot(hp, Wc1_ref[lig_nf:lig_nf + prot_nf, :]).astype(jnp.bfloat16)
    hpV_ref[0] = (_dot(hp, Wv1_ref[:prot_nf, :]) + bv1_ref[:]).astype(jnp.bfloat16)


def _xattn_body(hl_ref, xl_ref, hp_ref, xp_ref, lm_ref, pmt_ref,
                Wa1_ref, ba1_ref, Wa2_ref, ba2_ref, wa3_ref, ba3_ref,
                Wv1_ref, bv1_ref, Wv2_ref, bv2_ref,
                Wc1_ref, bc1_ref, wc2_ref, bc2_ref,
                hpA_in_ref, hpC_in_ref, hpV_in_ref,
                h_out_ref, x_out_ref):
    lig_nf = hl_ref.shape[2]
    prot_nf = hp_ref.shape[2]
    hl = hl_ref[0]
    xl = xl_ref[0]
    hp = hp_ref[0]
    xp = xp_ref[0]
    lm = lm_ref[0]
    pm2 = pmt_ref[0]

    xl2 = jnp.sum(xl * xl, axis=1, keepdims=True)
    xp2 = jnp.sum(xp * xp, axis=1, keepdims=True)
    d2 = jnp.maximum(xl2 + xp2.T - 2.0 * _dot(xl, xp.T, jax.lax.Precision.HIGHEST),
                     0.0)

    bf16 = jnp.bfloat16

    hlA = (_dot(hl, Wa1_ref[:lig_nf, :]) + ba1_ref[:]).astype(bf16)
    hlC = (_dot(hl, Wc1_ref[:lig_nf, :]) + bc1_ref[:]).astype(bf16)

    em2 = (d2 < _THRESH2).astype(jnp.float32)
    pe2 = pm2 * em2
    dist2 = jnp.sqrt(d2 + 1e-8)
    d2b = d2.astype(bf16)[:, :, None]

    a1 = _silu(hlA[:, None, :] + hpA_in_ref[0][None]
               + d2b * Wa1_ref[lig_nf + prot_nf:, :].astype(bf16)[None])
    a2 = _silu(_dot(a1, Wa2_ref[:].astype(bf16)).astype(bf16)
               + ba2_ref[:].astype(bf16)[None])
    la = _dot(a2, wa3_ref[:].astype(bf16))[:, :, 0]
    ae2 = jax.nn.sigmoid(la + ba3_ref[0, 0]) * pe2
    s = jnp.sum(ae2, axis=1, keepdims=True)

    v1 = _silu(hpV_in_ref[0][None] + d2b * Wv1_ref[prot_nf:, :].astype(bf16)[None])
    ae3 = ae2.astype(bf16)[:, :, None]
    t = jax.lax.dot_general(
        v1, ae3, (((1,), (1,)), ((0,), (0,))),
        preferred_element_type=jnp.float32)[:, :, 0]
    h_cross = (_dot(t, Wv2_ref[:]) + s * bv2_ref[:]) * (1.0 / _NORM) * lm

    c1 = _silu(hlC[:, None, :] + hpC_in_ref[0][None]
               + d2b * Wc1_ref[lig_nf + prot_nf:, :].astype(bf16)[None])
    lc = _dot(c1, wc2_ref[:].astype(bf16))[:, :, 0]
    g2 = jnp.tanh(lc + bc2_ref[0, 0]) * pe2 / (dist2 + 1e-8)
    gs = jnp.sum(g2, axis=1, keepdims=True)
    x_cross = (xl * gs - _dot(g2, xp)) * (1.0 / _NORM) * lm

    h_out_ref[0] = h_cross
    x_out_ref[0] = x_cross


@jax.jit
def kernel(h_ligand, x_ligand, h_protein, x_protein, ligand_mask, protein_mask,
           W_a1, b_a1, W_a2, b_a2, W_a3, b_a3,
           W_v1, b_v1, W_v2, b_v2, W_c1, b_c1, W_c2, b_c2):
    bs, nl, lig_nf = h_ligand.shape
    npn, prot_nf = h_protein.shape[1], h_protein.shape[2]
    hid = W_a2.shape[0]
    f32 = jnp.float32
    TL = 24
    nt = nl // TL

    pmt = jnp.transpose(protein_mask, (0, 2, 1))
    args = (
        h_ligand, x_ligand, h_protein, x_protein, ligand_mask, pmt,
        W_a1, b_a1.reshape(1, -1),
        W_a2, b_a2.reshape(1, -1), W_a3, b_a3.reshape(1, 1),
        W_v1, b_v1.reshape(1, -1),
        W_v2, b_v2.reshape(1, -1),
        W_c1, b_c1.reshape(1, -1), W_c2, b_c2.reshape(1, 1),
    )

    def lig_spec(a):
        shp = a.shape
        return pl.BlockSpec((1, TL) + shp[2:],
                            lambda b, t: (b, t) + (0,) * (len(shp) - 2))

    def batch_spec(a):
        shp = a.shape
        return pl.BlockSpec((1,) + shp[1:],
                            lambda b, t: (b,) + (0,) * (len(shp) - 1))

    def full_spec(a):
        shp = a.shape
        return pl.BlockSpec(shp, lambda b, t: (0,) * len(shp))

    in_specs = ([lig_spec(args[0]), lig_spec(args[1]),
                 batch_spec(args[2]), batch_spec(args[3]),
                 lig_spec(args[4]), batch_spec(args[5])]
                + [full_spec(a) for a in args[6:]])

    hpA, hpC, hpV = pl.pallas_call(
        _hp_proj_body,
        grid=(bs,),
        in_specs=[
            pl.BlockSpec((1, npn, prot_nf), lambda b: (b, 0, 0)),
            pl.BlockSpec(W_a1.shape, lambda b: (0, 0)),
            pl.BlockSpec(W_v1.shape, lambda b: (0, 0)),
            pl.BlockSpec((1, hid), lambda b: (0, 0)),
            pl.BlockSpec(W_c1.shape, lambda b: (0, 0)),
        ],
        out_specs=[pl.BlockSpec((1, npn, hid), lambda b: (b, 0, 0))] * 3,
        out_shape=[jax.ShapeDtypeStruct((bs, npn, hid), jnp.bfloat16)] * 3,
    )(h_protein, W_a1, W_v1, b_v1.reshape(1, -1), W_c1)

    hp_specs = [pl.BlockSpec((1, npn, hid), lambda b, t: (b, 0, 0))] * 3

    h_cross, x_cross = pl.pallas_call(
        _xattn_body,
        grid=(bs, nt),
        in_specs=in_specs + hp_specs,
        out_specs=[
            pl.BlockSpec((1, TL, lig_nf), lambda b, t: (b, t, 0)),
            pl.BlockSpec((1, TL, 3), lambda b, t: (b, t, 0)),
        ],
        out_shape=[
            jax.ShapeDtypeStruct((bs, nl, lig_nf), f32),
            jax.ShapeDtypeStruct((bs, nl, 3), f32),
        ],
    )(*args, hpA, hpC, hpV)
    return (h_cross, x_cross)


# speedup vs baseline: 1.0516x; 1.0516x over previous
import jax
import jax.numpy as jnp
from jax.experimental import pallas as pl
from jax.experimental.pallas import tpu as pltpu

_NORM = 100.0
_THRESH2 = 100.0


def _silu(x):
    h = 0.5 * x
    return h * jnp.tanh(h) + h


def _dot(a, b, prec=jax.lax.Precision.DEFAULT, out_t=jnp.float32):
    return jax.lax.dot_general(
        a, b, (((a.ndim - 1,), (0,)), ((), ())),
        preferred_element_type=out_t, precision=prec)


def _xattn_body(hl_ref, xl_ref, hp_ref, xp_ref, lm_ref, pmt_ref,
                Wa1_ref, ba1_ref, Wa2_ref, ba2_ref, wa3_ref, ba3_ref,
                Wv1_ref, bv1_ref, Wv2_ref, bv2_ref,
                Wc1_ref, bc1_ref, wc2_ref, bc2_ref,
                h_out_ref, x_out_ref,
                hpA_ref, hpC_ref, hpV_ref):
    lig_nf = hl_ref.shape[2]
    prot_nf = hp_ref.shape[2]
    hl = hl_ref[0]
    xl = xl_ref[0]
    hp = hp_ref[0]
    xp = xp_ref[0]
    lm = lm_ref[0]
    pm2 = pmt_ref[0]

    @pl.when(pl.program_id(1) == 0)
    def _():
        hpA_ref[:] = _dot(hp, Wa1_ref[lig_nf:lig_nf + prot_nf, :]).astype(jnp.bfloat16)
        hpC_ref[:] = _dot(hp, Wc1_ref[lig_nf:lig_nf + prot_nf, :]).astype(jnp.bfloat16)
        hpV_ref[:] = (_dot(hp, Wv1_ref[:prot_nf, :]) + bv1_ref[:]).astype(jnp.bfloat16)

    xl2 = jnp.sum(xl * xl, axis=1, keepdims=True)
    xp2 = jnp.sum(xp * xp, axis=1, keepdims=True)
    d2 = jnp.maximum(xl2 + xp2.T - 2.0 * _dot(xl, xp.T, jax.lax.Precision.HIGHEST),
                     0.0)

    bf16 = jnp.bfloat16

    hlA = (_dot(hl, Wa1_ref[:lig_nf, :]) + ba1_ref[:]).astype(bf16)
    hlC = (_dot(hl, Wc1_ref[:lig_nf, :]) + bc1_ref[:]).astype(bf16)

    em2 = (d2 < _THRESH2).astype(jnp.float32)
    pe2 = pm2 * em2
    dist2 = jnp.sqrt(d2 + 1e-8)
    d2b = d2.astype(bf16)[:, :, None]

    a1 = _silu(hlA[:, None, :] + hpA_ref[:][None]
               + d2b * Wa1_ref[lig_nf + prot_nf:, :].astype(bf16)[None])
    a2 = _silu(_dot(a1, Wa2_ref[:].astype(bf16)).astype(bf16)
               + ba2_ref[:].astype(bf16)[None])
    la = _dot(a2, wa3_ref[:].astype(bf16))[:, :, 0]
    ae2 = jax.nn.sigmoid(la + ba3_ref[0, 0]) * pe2
    s = jnp.sum(ae2, axis=1, keepdims=True)

    v1 = _silu(hpV_ref[:][None] + d2b * Wv1_ref[prot_nf:, :].astype(bf16)[None])
    ae3 = ae2.astype(bf16)[:, :, None]
    t = jax.lax.dot_general(
        v1, ae3, (((1,), (1,)), ((0,), (0,))),
        preferred_element_type=jnp.float32)[:, :, 0]
    h_cross = (_dot(t, Wv2_ref[:]) + s * bv2_ref[:]) * (1.0 / _NORM) * lm

    c1 = _silu(hlC[:, None, :] + hpC_ref[:][None]
               + d2b * Wc1_ref[lig_nf + prot_nf:, :].astype(bf16)[None])
    lc = _dot(c1, wc2_ref[:].astype(bf16))[:, :, 0]
    g2 = jnp.tanh(lc + bc2_ref[0, 0]) * pe2 / (dist2 + 1e-8)
    gs = jnp.sum(g2, axis=1, keepdims=True)
    x_cross = (xl * gs - _dot(g2, xp)) * (1.0 / _NORM) * lm

    h_out_ref[0] = h_cross
    x_out_ref[0] = x_cross


@jax.jit
def kernel(h_ligand, x_ligand, h_protein, x_protein, ligand_mask, protein_mask,
           W_a1, b_a1, W_a2, b_a2, W_a3, b_a3,
           W_v1, b_v1, W_v2, b_v2, W_c1, b_c1, W_c2, b_c2):
    bs, nl, lig_nf = h_ligand.shape
    npn, prot_nf = h_protein.shape[1], h_protein.shape[2]
    hid = W_a2.shape[0]
    f32 = jnp.float32
    TL = 24
    nt = nl // TL

    pmt = jnp.transpose(protein_mask, (0, 2, 1))
    args = (
        h_ligand, x_ligand, h_protein, x_protein, ligand_mask, pmt,
        W_a1, b_a1.reshape(1, -1),
        W_a2, b_a2.reshape(1, -1), W_a3, b_a3.reshape(1, 1),
        W_v1, b_v1.reshape(1, -1),
        W_v2, b_v2.reshape(1, -1),
        W_c1, b_c1.reshape(1, -1), W_c2, b_c2.reshape(1, 1),
    )

    def lig_spec(a):
        shp = a.shape
        return pl.BlockSpec((1, TL) + shp[2:],
                            lambda b, t: (b, t) + (0,) * (len(shp) - 2))

    def batch_spec(a):
        shp = a.shape
        return pl.BlockSpec((1,) + shp[1:],
                            lambda b, t: (b,) + (0,) * (len(shp) - 1))

    def full_spec(a):
        shp = a.shape
        return pl.BlockSpec(shp, lambda b, t: (0,) * len(shp))

    in_specs = ([lig_spec(args[0]), lig_spec(args[1]),
                 batch_spec(args[2]), batch_spec(args[3]),
                 lig_spec(args[4]), batch_spec(args[5])]
                + [full_spec(a) for a in args[6:]])

    h_cross, x_cross = pl.pallas_call(
        _xattn_body,
        grid=(bs, nt),
        in_specs=in_specs,
        out_specs=[
            pl.BlockSpec((1, TL, lig_nf), lambda b, t: (b, t, 0)),
            pl.BlockSpec((1, TL, 3), lambda b, t: (b, t, 0)),
        ],
        out_shape=[
            jax.ShapeDtypeStruct((bs, nl, lig_nf), f32),
            jax.ShapeDtypeStruct((bs, nl, 3), f32),
        ],
        scratch_shapes=[
            pltpu.VMEM((npn, hid), jnp.bfloat16),
            pltpu.VMEM((npn, hid), jnp.bfloat16),
            pltpu.VMEM((npn, hid), jnp.bfloat16),
        ],
    )(*args)
    return (h_cross, x_cross)


# speedup vs baseline: 1.0521x; 1.0005x over previous
import jax
import jax.numpy as jnp
from jax.experimental import pallas as pl
from jax.experimental.pallas import tpu as pltpu

_NORM = 100.0
_THRESH2 = 100.0


def _silu(x):
    h = 0.5 * x
    return h * jnp.tanh(h) + h


def _dot(a, b, prec=jax.lax.Precision.DEFAULT, out_t=jnp.float32):
    return jax.lax.dot_general(
        a, b, (((a.ndim - 1,), (0,)), ((), ())),
        preferred_element_type=out_t, precision=prec)


def _xattn_body(hl_ref, xl_ref, hp_ref, xp_ref, lm_ref, pmt_ref,
                Wa1_ref, ba1_ref, Wa2_ref, ba2_ref, wa3_ref, ba3_ref,
                Wv1_ref, bv1_ref, Wv2_ref, bv2_ref,
                Wc1_ref, bc1_ref, wc2_ref, bc2_ref,
                h_out_ref, x_out_ref,
                hpA_ref, hpC_ref, hpV_ref):
    lig_nf = hl_ref.shape[2]
    prot_nf = hp_ref.shape[2]
    hl = hl_ref[0]
    xl = xl_ref[0]
    hp = hp_ref[0]
    xp = xp_ref[0]
    lm = lm_ref[0]
    pm2 = pmt_ref[0]

    @pl.when(pl.program_id(1) == 0)
    def _():
        hpA_ref[:] = _dot(hp, Wa1_ref[lig_nf:lig_nf + prot_nf, :]).astype(jnp.bfloat16)
        hpC_ref[:] = _dot(hp, Wc1_ref[lig_nf:lig_nf + prot_nf, :]).astype(jnp.bfloat16)
        hpV_ref[:] = (_dot(hp, Wv1_ref[:prot_nf, :]) + bv1_ref[:]).astype(jnp.bfloat16)

    xl2 = jnp.sum(xl * xl, axis=1, keepdims=True)
    xp2 = jnp.sum(xp * xp, axis=1, keepdims=True)
    d2 = jnp.maximum(xl2 + xp2.T - 2.0 * _dot(xl, xp.T, jax.lax.Precision.HIGHEST),
                     0.0)

    bf16 = jnp.bfloat16

    hlA = (_dot(hl, Wa1_ref[:lig_nf, :]) + ba1_ref[:]).astype(bf16)
    hlC = (_dot(hl, Wc1_ref[:lig_nf, :]) + bc1_ref[:]).astype(bf16)

    em2 = (d2 < _THRESH2).astype(jnp.float32)
    pe2 = pm2 * em2
    dist2 = jnp.sqrt(d2 + 1e-8)
    d2b = d2.astype(bf16)[:, :, None]

    a1 = _silu(hlA[:, None, :] + hpA_ref[:][None]
               + d2b * Wa1_ref[lig_nf + prot_nf:, :].astype(bf16)[None])
    v1 = _silu(hpV_ref[:][None] + d2b * Wv1_ref[prot_nf:, :].astype(bf16)[None])
    a2 = _silu(_dot(a1, Wa2_ref[:].astype(bf16)).astype(bf16)
               + ba2_ref[:].astype(bf16)[None])
    la = _dot(a2, wa3_ref[:].astype(bf16))[:, :, 0]
    ae2 = jax.nn.sigmoid(la + ba3_ref[0, 0]) * pe2
    s = jnp.sum(ae2, axis=1, keepdims=True)

    ae3 = ae2.astype(bf16)[:, :, None]
    t = jax.lax.dot_general(
        v1, ae3, (((1,), (1,)), ((0,), (0,))),
        preferred_element_type=jnp.float32)[:, :, 0]
    h_cross = (_dot(t, Wv2_ref[:]) + s * bv2_ref[:]) * (1.0 / _NORM) * lm

    c1 = _silu(hlC[:, None, :] + hpC_ref[:][None]
               + d2b * Wc1_ref[lig_nf + prot_nf:, :].astype(bf16)[None])
    lc = _dot(c1, wc2_ref[:].astype(bf16))[:, :, 0]
    g2 = jnp.tanh(lc + bc2_ref[0, 0]) * pe2 / (dist2 + 1e-8)
    gs = jnp.sum(g2, axis=1, keepdims=True)
    x_cross = (xl * gs - _dot(g2, xp)) * (1.0 / _NORM) * lm

    h_out_ref[0] = h_cross
    x_out_ref[0] = x_cross


@jax.jit
def kernel(h_ligand, x_ligand, h_protein, x_protein, ligand_mask, protein_mask,
           W_a1, b_a1, W_a2, b_a2, W_a3, b_a3,
           W_v1, b_v1, W_v2, b_v2, W_c1, b_c1, W_c2, b_c2):
    bs, nl, lig_nf = h_ligand.shape
    npn, prot_nf = h_protein.shape[1], h_protein.shape[2]
    hid = W_a2.shape[0]
    f32 = jnp.float32
    TL = 24
    nt = nl // TL

    pmt = jnp.transpose(protein_mask, (0, 2, 1))
    args = (
        h_ligand, x_ligand, h_protein, x_protein, ligand_mask, pmt,
        W_a1, b_a1.reshape(1, -1),
        W_a2, b_a2.reshape(1, -1), W_a3, b_a3.reshape(1, 1),
        W_v1, b_v1.reshape(1, -1),
        W_v2, b_v2.reshape(1, -1),
        W_c1, b_c1.reshape(1, -1), W_c2, b_c2.reshape(1, 1),
    )

    def lig_spec(a):
        shp = a.shape
        return pl.BlockSpec((1, TL) + shp[2:],
                            lambda b, t: (b, t) + (0,) * (len(shp) - 2))

    def batch_spec(a):
        shp = a.shape
        return pl.BlockSpec((1,) + shp[1:],
                            lambda b, t: (b,) + (0,) * (len(shp) - 1))

    def full_spec(a):
        shp = a.shape
        return pl.BlockSpec(shp, lambda b, t: (0,) * len(shp))

    in_specs = ([lig_spec(args[0]), lig_spec(args[1]),
                 batch_spec(args[2]), batch_spec(args[3]),
                 lig_spec(args[4]), batch_spec(args[5])]
                + [full_spec(a) for a in args[6:]])

    h_cross, x_cross = pl.pallas_call(
        _xattn_body,
        grid=(bs, nt),
        in_specs=in_specs,
        out_specs=[
            pl.BlockSpec((1, TL, lig_nf), lambda b, t: (b, t, 0)),
            pl.BlockSpec((1, TL, 3), lambda b, t: (b, t, 0)),
        ],
        out_shape=[
            jax.ShapeDtypeStruct((bs, nl, lig_nf), f32),
            jax.ShapeDtypeStruct((bs, nl, 3), f32),
        ],
        scratch_shapes=[
            pltpu.VMEM((npn, hid), jnp.bfloat16),
            pltpu.VMEM((npn, hid), jnp.bfloat16),
            pltpu.VMEM((npn, hid), jnp.bfloat16),
        ],
    )(*args)
    return (h_cross, x_cross)
